# Initial kernel scaffold; baseline (speedup 1.0000x reference)
import jax
import jax.numpy as jnp
from jax.experimental import pallas as pl


def kernel(x, batch_ids, params):
    raise NotImplementedError("write your pallas kernel here")


# speedup vs baseline: 24.3857x; 24.3857x over previous
import jax
import jax.numpy as jnp
from jax.experimental import pallas as pl

_NUM_GRAPHS = 100
_K = 6
_D = 128


def _prelu(v, a):
    return jnp.where(v >= 0, v, a * v)


def _graph_kernel(xref,
                  pW1, pb1, pa1, pW2, pb2, pa2,
                  cW0, cb0, cW1, cb1, cW2, cb2, cW3, cb3,
                  qW1, qb1, qa1, qW2, qb2, qa2,
                  oref):
    xb = xref[0]
    n = xb.shape[0]
    f32 = jnp.float32

    x2 = xb * xb
    ones = jnp.ones((1, _D), f32)
    sq_col = jax.lax.dot_general(x2, ones, (((1,), (1,)), ((), ())),
                                 preferred_element_type=f32)
    sq_row = jax.lax.dot_general(ones, x2, (((1,), (1,)), ((), ())),
                                 preferred_element_type=f32)
    gram = jax.lax.dot_general(xb, xb, (((1,), (1,)), ((), ())),
                               preferred_element_type=f32)
    d = sq_col + sq_row - 2.0 * gram

    col = jax.lax.broadcasted_iota(jnp.int32, (n, n), 1)
    row = jax.lax.broadcasted_iota(jnp.int32, (n, n), 0)
    d = jnp.where(row == col, f32(1e10), d)

    A = jnp.zeros((n, n), f32)
    for _ in range(_K):
        m = jnp.min(d, axis=1, keepdims=True)
        sel = d == m
        A = A + sel.astype(f32)
        d = jnp.where(sel, f32(1e10), d)

    h = _prelu(jnp.dot(xb, pW1[...], preferred_element_type=f32) + pb1[0],
               pa1[0])
    h = _prelu(jnp.dot(h, pW2[...], preferred_element_type=f32) + pb2[0],
               pa2[0])

    for Wm, bm in ((cW0, cb0), (cW1, cb1), (cW2, cb2), (cW3, cb3)):
        msg = jnp.dot(h, Wm[...], preferred_element_type=f32) + bm[0]
        h = jnp.dot(A, msg, preferred_element_type=f32) + h

    h = _prelu(jnp.dot(h, qW1[...], preferred_element_type=f32) + qb1[0],
               qa1[0])
    h = _prelu(jnp.dot(h, qW2[...], preferred_element_type=f32) + qb2[0],
               qa2[0])

    oref[0] = jnp.sum(h, axis=0, keepdims=True)


def _head_kernel(pref, W1, b1, W2, b2, W3, b3, W4, b4, oref):
    p = pref[...]
    z = jnp.dot(p, W1[...], preferred_element_type=jnp.float32) + b1[0]
    z = jnp.where(z >= 0, z, 0.2 * z)
    z = jnp.dot(z, W2[...], preferred_element_type=jnp.float32) + b2[0]
    z = jnp.where(z >= 0, z, 0.2 * z)
    z = jnp.dot(z, W3[...], preferred_element_type=jnp.float32) + b3[0]
    z = jnp.where(z >= 0, z, 0.2 * z)
    z = jnp.dot(z, W4[...], preferred_element_type=jnp.float32) + b4[0]
    oref[...] = z


def kernel(x, batch_ids, params):
    n_total, d = x.shape
    nper = n_total // _NUM_GRAPHS
    xg = x.reshape(_NUM_GRAPHS, nper, d)

    def vec(name):
        return params[name].reshape(1, -1)

    weights = [
        params['pre_W1'], vec('pre_b1'), vec('pre_a1'),
        params['pre_W2'], vec('pre_b2'), vec('pre_a2'),
        params['conv0_Wm'], vec('conv0_bm'),
        params['conv1_Wm'], vec('conv1_bm'),
        params['conv2_Wm'], vec('conv2_bm'),
        params['conv3_Wm'], vec('conv3_bm'),
        params['post_W1'], vec('post_b1'), vec('post_a1'),
        params['post_W2'], vec('post_b2'), vec('post_a2'),
    ]

    in_specs = [pl.BlockSpec((1, nper, d), lambda i: (i, 0, 0))]
    for w in weights:
        in_specs.append(pl.BlockSpec(w.shape, lambda i: (0, 0)))

    pooled = pl.pallas_call(
        _graph_kernel,
        grid=(_NUM_GRAPHS,),
        in_specs=in_specs,
        out_specs=pl.BlockSpec((1, 1, d), lambda i: (i, 0, 0)),
        out_shape=jax.ShapeDtypeStruct((_NUM_GRAPHS, 1, d), jnp.float32),
    )(xg, *weights)

    pooled = pooled.reshape(_NUM_GRAPHS, d)

    z = pl.pallas_call(
        _head_kernel,
        out_shape=jax.ShapeDtypeStruct((_NUM_GRAPHS, 1), jnp.float32),
    )(pooled,
      params['hlv_W1'], vec('hlv_b1'),
      params['hlv_W2'], vec('hlv_b2'),
      params['hlv_W3'], vec('hlv_b3'),
      params['hlv_W4'], vec('hlv_b4'))
    return z


# speedup vs baseline: 24.4086x; 1.0009x over previous
import jax
import jax.numpy as jnp
from jax.experimental import pallas as pl
from jax.experimental.pallas import tpu as pltpu

_NUM_GRAPHS = 100
_K = 6
_D = 128


def _prelu(v, a):
    return jnp.where(v >= 0, v, a * v)


def _graph_kernel(xref,
                  pW1, pb1, pa1, pW2, pb2, pa2,
                  cW0, cb0, cW1, cb1, cW2, cb2, cW3, cb3,
                  qW1, qb1, qa1, qW2, qb2, qa2,
                  oref):
    xb = xref[0]
    n = xb.shape[0]
    f32 = jnp.float32

    x2 = xb * xb
    ones = jnp.ones((1, _D), f32)
    sq_col = jax.lax.dot_general(x2, ones, (((1,), (1,)), ((), ())),
                                 preferred_element_type=f32)
    sq_row = jax.lax.dot_general(ones, x2, (((1,), (1,)), ((), ())),
                                 preferred_element_type=f32)
    gram = jax.lax.dot_general(xb, xb, (((1,), (1,)), ((), ())),
                               preferred_element_type=f32)
    d = sq_col + sq_row - 2.0 * gram

    col = jax.lax.broadcasted_iota(jnp.int32, (n, n), 1)
    row = jax.lax.broadcasted_iota(jnp.int32, (n, n), 0)
    d = jnp.where(row == col, f32(1e10), d)

    A = jnp.zeros((n, n), f32)
    for _ in range(_K):
        m = jnp.min(d, axis=1, keepdims=True)
        sel = d == m
        A = A + sel.astype(f32)
        d = jnp.where(sel, f32(1e10), d)

    h = _prelu(jnp.dot(xb, pW1[...], preferred_element_type=f32) + pb1[0],
               pa1[0])
    h = _prelu(jnp.dot(h, pW2[...], preferred_element_type=f32) + pb2[0],
               pa2[0])

    for Wm, bm in ((cW0, cb0), (cW1, cb1), (cW2, cb2), (cW3, cb3)):
        msg = jnp.dot(h, Wm[...], preferred_element_type=f32) + bm[0]
        h = jnp.dot(A, msg, preferred_element_type=f32) + h

    h = _prelu(jnp.dot(h, qW1[...], preferred_element_type=f32) + qb1[0],
               qa1[0])
    h = _prelu(jnp.dot(h, qW2[...], preferred_element_type=f32) + qb2[0],
               qa2[0])

    oref[0] = jnp.sum(h, axis=0, keepdims=True)


def _head_kernel(pref, W1, b1, W2, b2, W3, b3, W4, b4, oref):
    p = pref[...]
    z = jnp.dot(p, W1[...], preferred_element_type=jnp.float32) + b1[0]
    z = jnp.where(z >= 0, z, 0.2 * z)
    z = jnp.dot(z, W2[...], preferred_element_type=jnp.float32) + b2[0]
    z = jnp.where(z >= 0, z, 0.2 * z)
    z = jnp.dot(z, W3[...], preferred_element_type=jnp.float32) + b3[0]
    z = jnp.where(z >= 0, z, 0.2 * z)
    z = jnp.dot(z, W4[...], preferred_element_type=jnp.float32) + b4[0]
    oref[...] = z


def kernel(x, batch_ids, params):
    n_total, d = x.shape
    nper = n_total // _NUM_GRAPHS
    xg = x.reshape(_NUM_GRAPHS, nper, d)

    def vec(name):
        return params[name].reshape(1, -1)

    weights = [
        params['pre_W1'], vec('pre_b1'), vec('pre_a1'),
        params['pre_W2'], vec('pre_b2'), vec('pre_a2'),
        params['conv0_Wm'], vec('conv0_bm'),
        params['conv1_Wm'], vec('conv1_bm'),
        params['conv2_Wm'], vec('conv2_bm'),
        params['conv3_Wm'], vec('conv3_bm'),
        params['post_W1'], vec('post_b1'), vec('post_a1'),
        params['post_W2'], vec('post_b2'), vec('post_a2'),
    ]

    in_specs = [pl.BlockSpec((1, nper, d), lambda i: (i, 0, 0))]
    for w in weights:
        in_specs.append(pl.BlockSpec(w.shape, lambda i: (0, 0)))

    pooled = pl.pallas_call(
        _graph_kernel,
        grid=(_NUM_GRAPHS,),
        in_specs=in_specs,
        out_specs=pl.BlockSpec((1, 1, d), lambda i: (i, 0, 0)),
        out_shape=jax.ShapeDtypeStruct((_NUM_GRAPHS, 1, d), jnp.float32),
        compiler_params=pltpu.CompilerParams(
            dimension_semantics=("parallel",)),
    )(xg, *weights)

    pooled = pooled.reshape(_NUM_GRAPHS, d)

    z = pl.pallas_call(
        _head_kernel,
        out_shape=jax.ShapeDtypeStruct((_NUM_GRAPHS, 1), jnp.float32),
    )(pooled,
      params['hlv_W1'], vec('hlv_b1'),
      params['hlv_W2'], vec('hlv_b2'),
      params['hlv_W3'], vec('hlv_b3'),
      params['hlv_W4'], vec('hlv_b4'))
    return z


# speedup vs baseline: 27.6973x; 1.1347x over previous
import jax
import jax.numpy as jnp
from jax.experimental import pallas as pl
from jax.experimental.pallas import tpu as pltpu

_NUM_GRAPHS = 100
_K = 6
_D = 128


def _prelu(v, a):
    return jnp.where(v >= 0, v, a * v)


def _graph_kernel(xref,
                  pW1, pb1, pa1, pW2, pb2, pa2,
                  cW0, cb0, cW1, cb1, cW2, cb2, cW3, cb3,
                  qW1, qb1, qa1, qW2, qb2, qa2,
                  oref):
    xb = xref[0]
    n = xb.shape[0]
    f32 = jnp.float32
    npad = (-n) % 8

    xp = jnp.concatenate([xb, jnp.full((npad, _D), 1e4, f32)], axis=0)
    x2 = xp * xp
    ones = jnp.ones((1, _D), f32)
    sq = jax.lax.dot_general(x2, ones, (((1,), (1,)), ((), ())),
                             preferred_element_type=f32)
    gram = jax.lax.dot_general(xp, xb, (((1,), (1,)), ((), ())),
                               preferred_element_type=f32)
    col = jax.lax.broadcasted_iota(jnp.int32, (n + npad, n), 1)
    row = jax.lax.broadcasted_iota(jnp.int32, (n + npad, n), 0)
    y = sq - 2.0 * gram
    y = jnp.where(row == col, f32(1e10), y)

    regs = [jnp.full((8, n), 1e30, f32) for _ in range(_K)]
    for g in range((n + npad) // 8):
        s = y[8 * g:8 * g + 8, :]
        for t in range(_K):
            keep = jnp.minimum(regs[t], s)
            s = jnp.maximum(regs[t], s)
            regs[t] = keep
    cand = jnp.concatenate(regs, axis=0)
    for _ in range(_K - 1):
        m = jnp.min(cand, axis=0, keepdims=True)
        cand = jnp.where(cand == m, f32(1e30), cand)
    thr = jnp.min(cand, axis=0, keepdims=True)

    A = (y <= thr).astype(f32)[:n, :]

    h = _prelu(jnp.dot(xb, pW1[...], preferred_element_type=f32) + pb1[0],
               pa1[0])
    h = _prelu(jnp.dot(h, pW2[...], preferred_element_type=f32) + pb2[0],
               pa2[0])

    for Wm, bm in ((cW0, cb0), (cW1, cb1), (cW2, cb2), (cW3, cb3)):
        msg = jnp.dot(h, Wm[...], preferred_element_type=f32) + bm[0]
        h = jax.lax.dot_general(A, msg, (((0,), (0,)), ((), ())),
                                preferred_element_type=f32) + h

    h = _prelu(jnp.dot(h, qW1[...], preferred_element_type=f32) + qb1[0],
               qa1[0])
    h = _prelu(jnp.dot(h, qW2[...], preferred_element_type=f32) + qb2[0],
               qa2[0])

    oref[0] = jnp.sum(h, axis=0, keepdims=True)


def _head_kernel(pref, W1, b1, W2, b2, W3, b3, W4, b4, oref):
    p = pref[...]
    z = jnp.dot(p, W1[...], preferred_element_type=jnp.float32) + b1[0]
    z = jnp.where(z >= 0, z, 0.2 * z)
    z = jnp.dot(z, W2[...], preferred_element_type=jnp.float32) + b2[0]
    z = jnp.where(z >= 0, z, 0.2 * z)
    z = jnp.dot(z, W3[...], preferred_element_type=jnp.float32) + b3[0]
    z = jnp.where(z >= 0, z, 0.2 * z)
    z = jnp.dot(z, W4[...], preferred_element_type=jnp.float32) + b4[0]
    oref[...] = z


def kernel(x, batch_ids, params):
    n_total, d = x.shape
    nper = n_total // _NUM_GRAPHS
    xg = x.reshape(_NUM_GRAPHS, nper, d)

    def vec(name):
        return params[name].reshape(1, -1)

    weights = [
        params['pre_W1'], vec('pre_b1'), vec('pre_a1'),
        params['pre_W2'], vec('pre_b2'), vec('pre_a2'),
        params['conv0_Wm'], vec('conv0_bm'),
        params['conv1_Wm'], vec('conv1_bm'),
        params['conv2_Wm'], vec('conv2_bm'),
        params['conv3_Wm'], vec('conv3_bm'),
        params['post_W1'], vec('post_b1'), vec('post_a1'),
        params['post_W2'], vec('post_b2'), vec('post_a2'),
    ]

    in_specs = [pl.BlockSpec((1, nper, d), lambda i: (i, 0, 0))]
    for w in weights:
        in_specs.append(pl.BlockSpec(w.shape, lambda i: (0, 0)))

    pooled = pl.pallas_call(
        _graph_kernel,
        grid=(_NUM_GRAPHS,),
        in_specs=in_specs,
        out_specs=pl.BlockSpec((1, 1, d), lambda i: (i, 0, 0)),
        out_shape=jax.ShapeDtypeStruct((_NUM_GRAPHS, 1, d), jnp.float32),
        compiler_params=pltpu.CompilerParams(
            dimension_semantics=("parallel",)),
    )(xg, *weights)

    pooled = pooled.reshape(_NUM_GRAPHS, d)

    z = pl.pallas_call(
        _head_kernel,
        out_shape=jax.ShapeDtypeStruct((_NUM_GRAPHS, 1), jnp.float32),
    )(pooled,
      params['hlv_W1'], vec('hlv_b1'),
      params['hlv_W2'], vec('hlv_b2'),
      params['hlv_W3'], vec('hlv_b3'),
      params['hlv_W4'], vec('hlv_b4'))
    return z


# speedup vs baseline: 29.8770x; 1.0787x over previous
import jax
import jax.numpy as jnp
from jax.experimental import pallas as pl
from jax.experimental.pallas import tpu as pltpu

_NUM_GRAPHS = 100
_K = 6
_D = 128
_GPB = 2


def _prelu(v, a):
    return jnp.where(v >= 0, v, a * v)


def _graph_kernel(xref,
                  pW1, pb1, pa1, pW2, pb2, pa2,
                  cW0, cb0, cW1, cb1, cW2, cb2, cW3, cb3,
                  qW1, qb1, qa1, qW2, qb2, qa2,
                  oref):
    for gi in range(_GPB):
        _one_graph(xref, gi,
                   pW1, pb1, pa1, pW2, pb2, pa2,
                   cW0, cb0, cW1, cb1, cW2, cb2, cW3, cb3,
                   qW1, qb1, qa1, qW2, qb2, qa2, oref)


def _one_graph(xref, gi,
               pW1, pb1, pa1, pW2, pb2, pa2,
               cW0, cb0, cW1, cb1, cW2, cb2, cW3, cb3,
               qW1, qb1, qa1, qW2, qb2, qa2, oref):
    xb = xref[gi]
    n = xb.shape[0]
    f32 = jnp.float32
    npad = (-n) % 8

    xp = jnp.concatenate([xb, jnp.full((npad, _D), 1e4, f32)], axis=0)
    x2 = xp * xp
    ones = jnp.ones((1, _D), f32)
    sq = jax.lax.dot_general(x2, ones, (((1,), (1,)), ((), ())),
                             preferred_element_type=f32)
    gram = jax.lax.dot_general(xp, xb, (((1,), (1,)), ((), ())),
                               preferred_element_type=f32)
    col = jax.lax.broadcasted_iota(jnp.int32, (n + npad, n), 1)
    row = jax.lax.broadcasted_iota(jnp.int32, (n + npad, n), 0)
    y = sq - 2.0 * gram
    y = jnp.where(row == col, f32(1e10), y)

    regs = [jnp.full((8, n), 1e30, f32) for _ in range(_K)]
    for g in range((n + npad) // 8):
        s = y[8 * g:8 * g + 8, :]
        for t in range(_K):
            keep = jnp.minimum(regs[t], s)
            s = jnp.maximum(regs[t], s)
            regs[t] = keep
    cand = jnp.concatenate(regs, axis=0)
    for _ in range(_K - 1):
        m = jnp.min(cand, axis=0, keepdims=True)
        cand = jnp.where(cand == m, f32(1e30), cand)
    thr = jnp.min(cand, axis=0, keepdims=True)

    A = (y <= thr).astype(f32)[:n, :]

    h = _prelu(jnp.dot(xb, pW1[...], preferred_element_type=f32) + pb1[0],
               pa1[0])
    h = _prelu(jnp.dot(h, pW2[...], preferred_element_type=f32) + pb2[0],
               pa2[0])

    for Wm, bm in ((cW0, cb0), (cW1, cb1), (cW2, cb2), (cW3, cb3)):
        msg = jnp.dot(h, Wm[...], preferred_element_type=f32) + bm[0]
        h = jax.lax.dot_general(A, msg, (((0,), (0,)), ((), ())),
                                preferred_element_type=f32) + h

    h = _prelu(jnp.dot(h, qW1[...], preferred_element_type=f32) + qb1[0],
               qa1[0])
    h = _prelu(jnp.dot(h, qW2[...], preferred_element_type=f32) + qb2[0],
               qa2[0])

    oref[gi] = jnp.sum(h, axis=0, keepdims=True)


def _head_kernel(pref, W1, b1, W2, b2, W3, b3, W4, b4, oref):
    p = pref[...]
    z = jnp.dot(p, W1[...], preferred_element_type=jnp.float32) + b1[0]
    z = jnp.where(z >= 0, z, 0.2 * z)
    z = jnp.dot(z, W2[...], preferred_element_type=jnp.float32) + b2[0]
    z = jnp.where(z >= 0, z, 0.2 * z)
    z = jnp.dot(z, W3[...], preferred_element_type=jnp.float32) + b3[0]
    z = jnp.where(z >= 0, z, 0.2 * z)
    z = jnp.dot(z, W4[...], preferred_element_type=jnp.float32) + b4[0]
    oref[...] = z


def kernel(x, batch_ids, params):
    n_total, d = x.shape
    nper = n_total // _NUM_GRAPHS
    xg = x.reshape(_NUM_GRAPHS, nper, d)

    def vec(name):
        return params[name].reshape(1, -1)

    weights = [
        params['pre_W1'], vec('pre_b1'), vec('pre_a1'),
        params['pre_W2'], vec('pre_b2'), vec('pre_a2'),
        params['conv0_Wm'], vec('conv0_bm'),
        params['conv1_Wm'], vec('conv1_bm'),
        params['conv2_Wm'], vec('conv2_bm'),
        params['conv3_Wm'], vec('conv3_bm'),
        params['post_W1'], vec('post_b1'), vec('post_a1'),
        params['post_W2'], vec('post_b2'), vec('post_a2'),
    ]

    in_specs = [pl.BlockSpec((_GPB, nper, d), lambda i: (i, 0, 0))]
    for w in weights:
        in_specs.append(pl.BlockSpec(w.shape, lambda i: (0, 0)))

    pooled = pl.pallas_call(
        _graph_kernel,
        grid=(_NUM_GRAPHS // _GPB,),
        in_specs=in_specs,
        out_specs=pl.BlockSpec((_GPB, 1, d), lambda i: (i, 0, 0)),
        out_shape=jax.ShapeDtypeStruct((_NUM_GRAPHS, 1, d), jnp.float32),
        compiler_params=pltpu.CompilerParams(
            dimension_semantics=("parallel",)),
    )(xg, *weights)

    pooled = pooled.reshape(_NUM_GRAPHS, d)

    z = pl.pallas_call(
        _head_kernel,
        out_shape=jax.ShapeDtypeStruct((_NUM_GRAPHS, 1), jnp.float32),
    )(pooled,
      params['hlv_W1'], vec('hlv_b1'),
      params['hlv_W2'], vec('hlv_b2'),
      params['hlv_W3'], vec('hlv_b3'),
      params['hlv_W4'], vec('hlv_b4'))
    return z


# speedup vs baseline: 30.7618x; 1.0296x over previous
import jax
import jax.numpy as jnp
from jax.experimental import pallas as pl
from jax.experimental.pallas import tpu as pltpu

_NUM_GRAPHS = 100
_K = 6
_D = 128
_GPB = 4


def _prelu(v, a):
    return jnp.where(v >= 0, v, a * v)


def _graph_kernel(xref,
                  pW1, pb1, pa1, pW2, pb2, pa2,
                  cW0, cb0, cW1, cb1, cW2, cb2, cW3, cb3,
                  qW1, qb1, qa1, qW2, qb2, qa2,
                  oref):
    for gi in range(_GPB):
        _one_graph(xref, gi,
                   pW1, pb1, pa1, pW2, pb2, pa2,
                   cW0, cb0, cW1, cb1, cW2, cb2, cW3, cb3,
                   qW1, qb1, qa1, qW2, qb2, qa2, oref)


def _one_graph(xref, gi,
               pW1, pb1, pa1, pW2, pb2, pa2,
               cW0, cb0, cW1, cb1, cW2, cb2, cW3, cb3,
               qW1, qb1, qa1, qW2, qb2, qa2, oref):
    xb = xref[gi]
    n = xb.shape[0]
    f32 = jnp.float32
    npad = (-n) % 8

    xp = jnp.concatenate([xb, jnp.full((npad, _D), 1e4, f32)], axis=0)
    x2 = xp * xp
    ones = jnp.ones((1, _D), f32)
    sq = jax.lax.dot_general(x2, ones, (((1,), (1,)), ((), ())),
                             preferred_element_type=f32)
    gram = jax.lax.dot_general(xp, xb, (((1,), (1,)), ((), ())),
                               preferred_element_type=f32)
    col = jax.lax.broadcasted_iota(jnp.int32, (n + npad, n), 1)
    row = jax.lax.broadcasted_iota(jnp.int32, (n + npad, n), 0)
    y = sq - 2.0 * gram
    y = jnp.where(row == col, f32(1e10), y)

    regs = [jnp.full((8, n), 1e30, f32) for _ in range(_K)]
    for g in range((n + npad) // 8):
        s = y[8 * g:8 * g + 8, :]
        for t in range(_K):
            keep = jnp.minimum(regs[t], s)
            s = jnp.maximum(regs[t], s)
            regs[t] = keep
    cand = jnp.concatenate(regs, axis=0)
    for _ in range(_K - 1):
        m = jnp.min(cand, axis=0, keepdims=True)
        cand = jnp.where(cand == m, f32(1e30), cand)
    thr = jnp.min(cand, axis=0, keepdims=True)

    A = (y <= thr).astype(f32)[:n, :]

    h = _prelu(jnp.dot(xb, pW1[...], preferred_element_type=f32) + pb1[0],
               pa1[0])
    h = _prelu(jnp.dot(h, pW2[...], preferred_element_type=f32) + pb2[0],
               pa2[0])

    for Wm, bm in ((cW0, cb0), (cW1, cb1), (cW2, cb2), (cW3, cb3)):
        msg = jnp.dot(h, Wm[...], preferred_element_type=f32) + bm[0]
        h = jax.lax.dot_general(A, msg, (((0,), (0,)), ((), ())),
                                preferred_element_type=f32) + h

    h = _prelu(jnp.dot(h, qW1[...], preferred_element_type=f32) + qb1[0],
               qa1[0])
    h = _prelu(jnp.dot(h, qW2[...], preferred_element_type=f32) + qb2[0],
               qa2[0])

    oref[gi] = jnp.sum(h, axis=0, keepdims=True)


def _head_kernel(pref, W1, b1, W2, b2, W3, b3, W4, b4, oref):
    p = pref[...]
    z = jnp.dot(p, W1[...], preferred_element_type=jnp.float32) + b1[0]
    z = jnp.where(z >= 0, z, 0.2 * z)
    z = jnp.dot(z, W2[...], preferred_element_type=jnp.float32) + b2[0]
    z = jnp.where(z >= 0, z, 0.2 * z)
    z = jnp.dot(z, W3[...], preferred_element_type=jnp.float32) + b3[0]
    z = jnp.where(z >= 0, z, 0.2 * z)
    z = jnp.dot(z, W4[...], preferred_element_type=jnp.float32) + b4[0]
    oref[...] = z


def kernel(x, batch_ids, params):
    n_total, d = x.shape
    nper = n_total // _NUM_GRAPHS
    xg = x.reshape(_NUM_GRAPHS, nper, d)

    def vec(name):
        return params[name].reshape(1, -1)

    weights = [
        params['pre_W1'], vec('pre_b1'), vec('pre_a1'),
        params['pre_W2'], vec('pre_b2'), vec('pre_a2'),
        params['conv0_Wm'], vec('conv0_bm'),
        params['conv1_Wm'], vec('conv1_bm'),
        params['conv2_Wm'], vec('conv2_bm'),
        params['conv3_Wm'], vec('conv3_bm'),
        params['post_W1'], vec('post_b1'), vec('post_a1'),
        params['post_W2'], vec('post_b2'), vec('post_a2'),
    ]

    in_specs = [pl.BlockSpec((_GPB, nper, d), lambda i: (i, 0, 0))]
    for w in weights:
        in_specs.append(pl.BlockSpec(w.shape, lambda i: (0, 0)))

    pooled = pl.pallas_call(
        _graph_kernel,
        grid=(_NUM_GRAPHS // _GPB,),
        in_specs=in_specs,
        out_specs=pl.BlockSpec((_GPB, 1, d), lambda i: (i, 0, 0)),
        out_shape=jax.ShapeDtypeStruct((_NUM_GRAPHS, 1, d), jnp.float32),
        compiler_params=pltpu.CompilerParams(
            dimension_semantics=("parallel",)),
    )(xg, *weights)

    pooled = pooled.reshape(_NUM_GRAPHS, d)

    z = pl.pallas_call(
        _head_kernel,
        out_shape=jax.ShapeDtypeStruct((_NUM_GRAPHS, 1), jnp.float32),
    )(pooled,
      params['hlv_W1'], vec('hlv_b1'),
      params['hlv_W2'], vec('hlv_b2'),
      params['hlv_W3'], vec('hlv_b3'),
      params['hlv_W4'], vec('hlv_b4'))
    return z


# speedup vs baseline: 31.0312x; 1.0088x over previous
import jax
import jax.numpy as jnp
from jax.experimental import pallas as pl
from jax.experimental.pallas import tpu as pltpu

_NUM_GRAPHS = 100
_K = 6
_D = 128
_GPB = 5


def _prelu(v, a):
    return jnp.where(v >= 0, v, a * v)


def _graph_kernel(xref,
                  pW1, pb1, pa1, pW2, pb2, pa2,
                  cW0, cb0, cW1, cb1, cW2, cb2, cW3, cb3,
                  qW1, qb1, qa1, qW2, qb2, qa2,
                  oref):
    for gi in range(_GPB):
        _one_graph(xref, gi,
                   pW1, pb1, pa1, pW2, pb2, pa2,
                   cW0, cb0, cW1, cb1, cW2, cb2, cW3, cb3,
                   qW1, qb1, qa1, qW2, qb2, qa2, oref)


def _one_graph(xref, gi,
               pW1, pb1, pa1, pW2, pb2, pa2,
               cW0, cb0, cW1, cb1, cW2, cb2, cW3, cb3,
               qW1, qb1, qa1, qW2, qb2, qa2, oref):
    xb = xref[gi]
    n = xb.shape[0]
    f32 = jnp.float32
    npad = (-n) % 8

    xp = jnp.concatenate([xb, jnp.full((npad, _D), 1e4, f32)], axis=0)
    x2 = xp * xp
    ones = jnp.ones((1, _D), f32)
    sq = jax.lax.dot_general(x2, ones, (((1,), (1,)), ((), ())),
                             preferred_element_type=f32)
    gram = jax.lax.dot_general(xp, xb, (((1,), (1,)), ((), ())),
                               preferred_element_type=f32)
    col = jax.lax.broadcasted_iota(jnp.int32, (n + npad, n), 1)
    row = jax.lax.broadcasted_iota(jnp.int32, (n + npad, n), 0)
    y = sq - 2.0 * gram
    y = jnp.where(row == col, f32(1e10), y)

    regs = [jnp.full((8, n), 1e30, f32) for _ in range(_K)]
    for g in range((n + npad) // 8):
        s = y[8 * g:8 * g + 8, :]
        for t in range(_K):
            keep = jnp.minimum(regs[t], s)
            s = jnp.maximum(regs[t], s)
            regs[t] = keep
    cand = jnp.concatenate(regs, axis=0)
    for _ in range(_K - 1):
        m = jnp.min(cand, axis=0, keepdims=True)
        cand = jnp.where(cand == m, f32(1e30), cand)
    thr = jnp.min(cand, axis=0, keepdims=True)

    A = (y <= thr).astype(f32)[:n, :]

    h = _prelu(jnp.dot(xb, pW1[...], preferred_element_type=f32) + pb1[0],
               pa1[0])
    h = _prelu(jnp.dot(h, pW2[...], preferred_element_type=f32) + pb2[0],
               pa2[0])

    for Wm, bm in ((cW0, cb0), (cW1, cb1), (cW2, cb2), (cW3, cb3)):
        msg = jnp.dot(h, Wm[...], preferred_element_type=f32) + bm[0]
        h = jax.lax.dot_general(A, msg, (((0,), (0,)), ((), ())),
                                preferred_element_type=f32) + h

    h = _prelu(jnp.dot(h, qW1[...], preferred_element_type=f32) + qb1[0],
               qa1[0])
    h = _prelu(jnp.dot(h, qW2[...], preferred_element_type=f32) + qb2[0],
               qa2[0])

    oref[gi] = jnp.sum(h, axis=0, keepdims=True)


def _head_kernel(pref, W1, b1, W2, b2, W3, b3, W4, b4, oref):
    p = pref[...]
    z = jnp.dot(p, W1[...], preferred_element_type=jnp.float32) + b1[0]
    z = jnp.where(z >= 0, z, 0.2 * z)
    z = jnp.dot(z, W2[...], preferred_element_type=jnp.float32) + b2[0]
    z = jnp.where(z >= 0, z, 0.2 * z)
    z = jnp.dot(z, W3[...], preferred_element_type=jnp.float32) + b3[0]
    z = jnp.where(z >= 0, z, 0.2 * z)
    z = jnp.dot(z, W4[...], preferred_element_type=jnp.float32) + b4[0]
    oref[...] = z


def kernel(x, batch_ids, params):
    n_total, d = x.shape
    nper = n_total // _NUM_GRAPHS
    xg = x.reshape(_NUM_GRAPHS, nper, d)

    def vec(name):
        return params[name].reshape(1, -1)

    weights = [
        params['pre_W1'], vec('pre_b1'), vec('pre_a1'),
        params['pre_W2'], vec('pre_b2'), vec('pre_a2'),
        params['conv0_Wm'], vec('conv0_bm'),
        params['conv1_Wm'], vec('conv1_bm'),
        params['conv2_Wm'], vec('conv2_bm'),
        params['conv3_Wm'], vec('conv3_bm'),
        params['post_W1'], vec('post_b1'), vec('post_a1'),
        params['post_W2'], vec('post_b2'), vec('post_a2'),
    ]

    in_specs = [pl.BlockSpec((_GPB, nper, d), lambda i: (i, 0, 0))]
    for w in weights:
        in_specs.append(pl.BlockSpec(w.shape, lambda i: (0, 0)))

    pooled = pl.pallas_call(
        _graph_kernel,
        grid=(_NUM_GRAPHS // _GPB,),
        in_specs=in_specs,
        out_specs=pl.BlockSpec((_GPB, 1, d), lambda i: (i, 0, 0)),
        out_shape=jax.ShapeDtypeStruct((_NUM_GRAPHS, 1, d), jnp.float32),
        compiler_params=pltpu.CompilerParams(
            dimension_semantics=("parallel",)),
    )(xg, *weights)

    pooled = pooled.reshape(_NUM_GRAPHS, d)

    z = pl.pallas_call(
        _head_kernel,
        out_shape=jax.ShapeDtypeStruct((_NUM_GRAPHS, 1), jnp.float32),
    )(pooled,
      params['hlv_W1'], vec('hlv_b1'),
      params['hlv_W2'], vec('hlv_b2'),
      params['hlv_W3'], vec('hlv_b3'),
      params['hlv_W4'], vec('hlv_b4'))
    return z


# speedup vs baseline: 31.4066x; 1.0121x over previous
import jax
import jax.numpy as jnp
from jax.experimental import pallas as pl
from jax.experimental.pallas import tpu as pltpu

_NUM_GRAPHS = 100
_K = 6
_D = 128
_GPB = 10


def _prelu(v, a):
    return jnp.where(v >= 0, v, a * v)


def _graph_kernel(xref,
                  pW1, pb1, pa1, pW2, pb2, pa2,
                  cW0, cb0, cW1, cb1, cW2, cb2, cW3, cb3,
                  qW1, qb1, qa1, qW2, qb2, qa2,
                  oref):
    for gi in range(_GPB):
        _one_graph(xref, gi,
                   pW1, pb1, pa1, pW2, pb2, pa2,
                   cW0, cb0, cW1, cb1, cW2, cb2, cW3, cb3,
                   qW1, qb1, qa1, qW2, qb2, qa2, oref)


def _one_graph(xref, gi,
               pW1, pb1, pa1, pW2, pb2, pa2,
               cW0, cb0, cW1, cb1, cW2, cb2, cW3, cb3,
               qW1, qb1, qa1, qW2, qb2, qa2, oref):
    xb = xref[gi]
    n = xb.shape[0]
    f32 = jnp.float32
    npad = (-n) % 8

    xp = jnp.concatenate([xb, jnp.full((npad, _D), 1e4, f32)], axis=0)
    x2 = xp * xp
    ones = jnp.ones((1, _D), f32)
    sq = jax.lax.dot_general(x2, ones, (((1,), (1,)), ((), ())),
                             preferred_element_type=f32)
    gram = jax.lax.dot_general(xp, xb, (((1,), (1,)), ((), ())),
                               preferred_element_type=f32)
    col = jax.lax.broadcasted_iota(jnp.int32, (n + npad, n), 1)
    row = jax.lax.broadcasted_iota(jnp.int32, (n + npad, n), 0)
    y = sq - 2.0 * gram
    y = jnp.where(row == col, f32(1e10), y)

    regs = [jnp.full((8, n), 1e30, f32) for _ in range(_K)]
    for g in range((n + npad) // 8):
        s = y[8 * g:8 * g + 8, :]
        for t in range(_K):
            keep = jnp.minimum(regs[t], s)
            s = jnp.maximum(regs[t], s)
            regs[t] = keep
    cand = jnp.concatenate(regs, axis=0)
    for _ in range(_K - 1):
        m = jnp.min(cand, axis=0, keepdims=True)
        cand = jnp.where(cand == m, f32(1e30), cand)
    thr = jnp.min(cand, axis=0, keepdims=True)

    A = (y <= thr).astype(f32)[:n, :]

    h = _prelu(jnp.dot(xb, pW1[...], preferred_element_type=f32) + pb1[0],
               pa1[0])
    h = _prelu(jnp.dot(h, pW2[...], preferred_element_type=f32) + pb2[0],
               pa2[0])

    for Wm, bm in ((cW0, cb0), (cW1, cb1), (cW2, cb2), (cW3, cb3)):
        msg = jnp.dot(h, Wm[...], preferred_element_type=f32) + bm[0]
        h = jax.lax.dot_general(A, msg, (((0,), (0,)), ((), ())),
                                preferred_element_type=f32) + h

    h = _prelu(jnp.dot(h, qW1[...], preferred_element_type=f32) + qb1[0],
               qa1[0])
    h = _prelu(jnp.dot(h, qW2[...], preferred_element_type=f32) + qb2[0],
               qa2[0])

    oref[gi] = jnp.sum(h, axis=0, keepdims=True)


def _head_kernel(pref, W1, b1, W2, b2, W3, b3, W4, b4, oref):
    p = pref[...]
    z = jnp.dot(p, W1[...], preferred_element_type=jnp.float32) + b1[0]
    z = jnp.where(z >= 0, z, 0.2 * z)
    z = jnp.dot(z, W2[...], preferred_element_type=jnp.float32) + b2[0]
    z = jnp.where(z >= 0, z, 0.2 * z)
    z = jnp.dot(z, W3[...], preferred_element_type=jnp.float32) + b3[0]
    z = jnp.where(z >= 0, z, 0.2 * z)
    z = jnp.dot(z, W4[...], preferred_element_type=jnp.float32) + b4[0]
    oref[...] = z


def kernel(x, batch_ids, params):
    n_total, d = x.shape
    nper = n_total // _NUM_GRAPHS
    xg = x.reshape(_NUM_GRAPHS, nper, d)

    def vec(name):
        return params[name].reshape(1, -1)

    weights = [
        params['pre_W1'], vec('pre_b1'), vec('pre_a1'),
        params['pre_W2'], vec('pre_b2'), vec('pre_a2'),
        params['conv0_Wm'], vec('conv0_bm'),
        params['conv1_Wm'], vec('conv1_bm'),
        params['conv2_Wm'], vec('conv2_bm'),
        params['conv3_Wm'], vec('conv3_bm'),
        params['post_W1'], vec('post_b1'), vec('post_a1'),
        params['post_W2'], vec('post_b2'), vec('post_a2'),
    ]

    in_specs = [pl.BlockSpec((_GPB, nper, d), lambda i: (i, 0, 0))]
    for w in weights:
        in_specs.append(pl.BlockSpec(w.shape, lambda i: (0, 0)))

    pooled = pl.pallas_call(
        _graph_kernel,
        grid=(_NUM_GRAPHS // _GPB,),
        in_specs=in_specs,
        out_specs=pl.BlockSpec((_GPB, 1, d), lambda i: (i, 0, 0)),
        out_shape=jax.ShapeDtypeStruct((_NUM_GRAPHS, 1, d), jnp.float32),
        compiler_params=pltpu.CompilerParams(
            dimension_semantics=("parallel",)),
    )(xg, *weights)

    pooled = pooled.reshape(_NUM_GRAPHS, d)

    z = pl.pallas_call(
        _head_kernel,
        out_shape=jax.ShapeDtypeStruct((_NUM_GRAPHS, 1), jnp.float32),
    )(pooled,
      params['hlv_W1'], vec('hlv_b1'),
      params['hlv_W2'], vec('hlv_b2'),
      params['hlv_W3'], vec('hlv_b3'),
      params['hlv_W4'], vec('hlv_b4'))
    return z
